# baseline (speedup 1.0000x reference)
import functools

import jax
import jax.numpy as jnp
from jax import lax
from jax.experimental import pallas as pl
from jax.experimental.pallas import tpu as pltpu
from jax.experimental.pallas import tpu_sc as plsc

NC = 2
NS = 16
NW = NC * NS
LANES = 16


NBUF = 2
UNROLL = 5


def _make_pool(vocab, embed, batch, seq):
  rows_per_w = batch // NW
  toks_per_w = rows_per_w * seq
  nreg = embed // LANES
  chunks = []
  off = 0
  while off < seq:
    n = min(128, seq - off)
    if n < seq - off and n % 8:
      n -= n % 8
    chunks.append((off, n))
    off += n
  mesh = plsc.VectorSubcoreMesh(
      core_axis_name="c", subcore_axis_name="s",
      num_cores=NC, num_subcores=NS)

  def body(x_hbm, emb_hbm, out_hbm, idx_v, buf_v, acc_v, *sems):
    wid = lax.axis_index("s") * NC + lax.axis_index("c")
    pltpu.sync_copy(x_hbm.at[pl.ds(wid * toks_per_w, toks_per_w)], idx_v)

    def fire(b, p):
      tbase = pl.multiple_of(b * seq, 8)
      for off, n in chunks:
        pltpu.async_copy(emb_hbm.at[idx_v.at[pl.ds(tbase + off, n)]],
                         buf_v.at[p, pl.ds(off, n)], sems[p])

    def wait(p):
      for off, n in chunks:
        pltpu.make_async_copy(emb_hbm.at[idx_v.at[pl.ds(off, n)]],
                              buf_v.at[p, pl.ds(off, n)], sems[p]).wait()

    for p in range(NBUF):
      fire(p, p)

    def accum(p, b):
      def tok(t, acc):
        for u in range(UNROLL):
          acc = tuple(
              acc[k] + buf_v[p, UNROLL * t + u, pl.ds(LANES * k, LANES)]
              for k in range(nreg))
        return acc
      acc = tuple(jnp.zeros((LANES,), jnp.float32) for _ in range(nreg))
      acc = lax.fori_loop(0, seq // UNROLL, tok, acc)
      for k in range(nreg):
        acc_v[b, pl.ds(LANES * k, LANES)] = acc[k]

    def step(i, carry):
      for p in range(NBUF):
        b = NBUF * i + p
        wait(p)
        accum(p, b)

        @pl.when(b + NBUF < rows_per_w)
        def _():
          fire(b + NBUF, p)
      return carry

    lax.fori_loop(0, rows_per_w // NBUF, step, 0)
    pltpu.sync_copy(acc_v, out_hbm.at[pl.ds(wid * rows_per_w, rows_per_w)])

  return pl.kernel(
      body,
      out_type=jax.ShapeDtypeStruct((batch, embed), jnp.float32),
      mesh=mesh,
      scratch_types=[
          pltpu.VMEM((toks_per_w,), jnp.int32),
          pltpu.VMEM((NBUF, seq, embed), jnp.float32),
          pltpu.VMEM((rows_per_w, embed), jnp.float32),
      ] + [pltpu.SemaphoreType.DMA] * NBUF,
  )


def _mlp_body(inv_l, p_ref, w1_ref, b1_ref, w2_ref, b2_ref, o_ref):
  pooled = p_ref[:] * inv_l
  h = jnp.maximum(
      jnp.dot(pooled, w1_ref[:], preferred_element_type=jnp.float32)
      + b1_ref[:][None, :], 0.0)
  o_ref[:] = (
      jnp.dot(h, w2_ref[:], preferred_element_type=jnp.float32)
      + b2_ref[:][None, :])


def kernel(x, embedding, W1, b1, W2, b2):
  batch, seq = x.shape
  vocab, embed = embedding.shape
  hidden = W1.shape[1]
  ncls = W2.shape[1]
  del hidden

  xr = x.astype(jnp.int32).reshape(batch * seq)
  pool = _make_pool(vocab, embed, batch, seq)
  pooled_sum = pool(xr, embedding)

  mlp = pl.pallas_call(
      functools.partial(_mlp_body, 1.0 / seq),
      out_shape=jax.ShapeDtypeStruct((batch, ncls), jnp.float32),
  )
  return mlp(pooled_sum, W1, b1, W2, b2)


# speedup vs baseline: 1.2411x; 1.2411x over previous
import functools

import jax
import jax.numpy as jnp
from jax import lax
from jax.experimental import pallas as pl
from jax.experimental.pallas import tpu as pltpu
from jax.experimental.pallas import tpu_sc as plsc

NC = 2
NS = 16
NW = NC * NS
LANES = 16


NBUF = 3
UNROLL = 4


def _make_pool(vocab, embed, batch, seq_chunks, chunk):
  rows_per_w = batch // NW
  nreg = embed // LANES
  mesh = plsc.VectorSubcoreMesh(
      core_axis_name="c", subcore_axis_name="s",
      num_cores=NC, num_subcores=NS)

  def body(x_hbm, emb_hbm, out_hbm, idx_v, buf_v, acc_v, *sems):
    wid = lax.axis_index("s") * NC + lax.axis_index("c")
    base = wid * rows_per_w
    pltpu.sync_copy(x_hbm.at[pl.ds(base, rows_per_w)], idx_v)

    def fire(b, p):
      for j in range(seq_chunks):
        pltpu.async_copy(emb_hbm.at[idx_v.at[b, j]], buf_v.at[p, j], sems[p])

    def wait(p):
      for j in range(seq_chunks):
        pltpu.make_async_copy(
            emb_hbm.at[idx_v.at[0, j]], buf_v.at[p, j], sems[p]).wait()

    for p in range(NBUF):
      fire(p, p)

    def accum(p, b):
      def tok(j):
        def f(t, acc):
          for u in range(UNROLL):
            acc = tuple(
                acc[k] + buf_v[p, j, UNROLL * t + u, pl.ds(LANES * k, LANES)]
                for k in range(nreg))
          return acc
        return f
      acc = tuple(jnp.zeros((LANES,), jnp.float32) for _ in range(nreg))
      for j in range(seq_chunks):
        acc = lax.fori_loop(0, chunk // UNROLL, tok(j), acc)
      for k in range(nreg):
        acc_v[b, pl.ds(LANES * k, LANES)] = acc[k]

    main_iters = rows_per_w // NBUF

    def step(i, carry):
      for p in range(NBUF):
        b = NBUF * i + p
        wait(p)
        accum(p, b)

        @pl.when(b + NBUF < rows_per_w)
        def _():
          fire(b + NBUF, p)
      return carry

    lax.fori_loop(0, main_iters, step, 0)
    for b in range(NBUF * main_iters, rows_per_w):
      p = b % NBUF
      wait(p)
      accum(p, b)
    pltpu.sync_copy(acc_v, out_hbm.at[pl.ds(base, rows_per_w)])

  return pl.kernel(
      body,
      out_type=jax.ShapeDtypeStruct((batch, embed), jnp.float32),
      mesh=mesh,
      scratch_types=[
          pltpu.VMEM((rows_per_w, seq_chunks, chunk), jnp.int32),
          pltpu.VMEM((NBUF, seq_chunks, chunk, embed), jnp.float32),
          pltpu.VMEM((rows_per_w, embed), jnp.float32),
      ] + [pltpu.SemaphoreType.DMA] * NBUF,
  )


def _mlp_body(inv_l, p_ref, w1_ref, b1_ref, w2_ref, b2_ref, o_ref):
  pooled = p_ref[:] * inv_l
  h = jnp.maximum(
      jnp.dot(pooled, w1_ref[:], preferred_element_type=jnp.float32)
      + b1_ref[:][None, :], 0.0)
  o_ref[:] = (
      jnp.dot(h, w2_ref[:], preferred_element_type=jnp.float32)
      + b2_ref[:][None, :])


def kernel(x, embedding, W1, b1, W2, b2):
  batch, seq = x.shape
  vocab, embed = embedding.shape
  hidden = W1.shape[1]
  ncls = W2.shape[1]
  del hidden
  chunk = 100
  seq_chunks = seq // chunk

  xr = x.astype(jnp.int32).reshape(batch, seq_chunks, chunk)
  pool = _make_pool(vocab, embed, batch, seq_chunks, chunk)
  pooled_sum = pool(xr, embedding)

  mlp = pl.pallas_call(
      functools.partial(_mlp_body, 1.0 / seq),
      out_shape=jax.ShapeDtypeStruct((batch, ncls), jnp.float32),
  )
  return mlp(pooled_sum, W1, b1, W2, b2)


# speedup vs baseline: 1.2600x; 1.0152x over previous
import functools

import jax
import jax.numpy as jnp
from jax import lax
from jax.experimental import pallas as pl
from jax.experimental.pallas import tpu as pltpu
from jax.experimental.pallas import tpu_sc as plsc

NC = 2
NS = 16
NW = NC * NS
LANES = 16


NBUF = 3
UNROLL = 4


def _make_pool(vocab, embed, batch, seq_chunks, chunk):
  rows_per_w = batch // NW
  nreg = embed // LANES
  mesh = plsc.VectorSubcoreMesh(
      core_axis_name="c", subcore_axis_name="s",
      num_cores=NC, num_subcores=NS)

  def body(x_hbm, emb_hbm, out_hbm, idx_v, buf_v, acc_v, *sems):
    wid = lax.axis_index("s") * NC + lax.axis_index("c")
    base = wid * rows_per_w
    pltpu.sync_copy(x_hbm.at[pl.ds(base, rows_per_w)], idx_v)

    def fire(b, p):
      for j in range(seq_chunks):
        pltpu.async_copy(emb_hbm.at[idx_v.at[b, j]], buf_v.at[p, j], sems[p])

    def wait(p):
      for j in range(seq_chunks):
        pltpu.make_async_copy(
            emb_hbm.at[idx_v.at[0, j]], buf_v.at[p, j], sems[p]).wait()

    for p in range(NBUF):
      fire(p, p)

    def accum(p, b):
      def tok(j):
        def f(t, acc):
          for u in range(UNROLL):
            acc = tuple(
                acc[k] + buf_v[p, j, UNROLL * t + u, pl.ds(LANES * k, LANES)]
                for k in range(nreg))
          return acc
        return f
      acc = tuple(jnp.zeros((LANES,), jnp.float32) for _ in range(nreg))
      for j in range(seq_chunks):
        acc = lax.fori_loop(0, chunk // UNROLL, tok(j), acc)
      for k in range(nreg):
        acc_v[b, pl.ds(LANES * k, LANES)] = acc[k]

    main_iters = rows_per_w // NBUF

    def step(i, carry):
      for p in range(NBUF):
        b = NBUF * i + p
        wait(p)
        accum(p, b)

        @pl.when(b + NBUF < rows_per_w)
        def _():
          fire(b + NBUF, p)
      return carry

    lax.fori_loop(0, main_iters, step, 0)
    for b in range(NBUF * main_iters, rows_per_w):
      p = b % NBUF
      wait(p)
      accum(p, b)
    pltpu.sync_copy(acc_v, out_hbm.at[pl.ds(base, rows_per_w)])

  return pl.kernel(
      body,
      out_type=jax.ShapeDtypeStruct((batch, embed), jnp.float32),
      mesh=mesh,
      scratch_types=[
          pltpu.VMEM((rows_per_w, seq_chunks, chunk), jnp.int32),
          pltpu.VMEM((NBUF, seq_chunks, chunk, embed), jnp.float32),
          pltpu.VMEM((rows_per_w, embed), jnp.float32),
      ] + [pltpu.SemaphoreType.DMA] * NBUF,
  )


def _mlp_body(inv_l, p_ref, w1_ref, b1_ref, w2_ref, b2_ref, o_ref):
  pooled = p_ref[:] * inv_l
  h = jnp.maximum(
      jnp.dot(pooled, w1_ref[:], preferred_element_type=jnp.float32)
      + b1_ref[:][None, :], 0.0)
  out_t = lax.dot_general(
      w2_ref[:], h, (((0,), (1,)), ((), ())),
      preferred_element_type=jnp.float32)
  o_ref[:] = out_t + b2_ref[:][:, None]


def kernel(x, embedding, W1, b1, W2, b2):
  batch, seq = x.shape
  vocab, embed = embedding.shape
  hidden = W1.shape[1]
  ncls = W2.shape[1]
  del hidden
  chunk = 100
  seq_chunks = seq // chunk

  xr = x.astype(jnp.int32).reshape(batch, seq_chunks, chunk)
  pool = _make_pool(vocab, embed, batch, seq_chunks, chunk)
  pooled_sum = pool(xr, embedding)

  mlp = pl.pallas_call(
      functools.partial(_mlp_body, 1.0 / seq),
      out_shape=jax.ShapeDtypeStruct((ncls, batch), jnp.float32),
  )
  return mlp(pooled_sum, W1, b1, W2, b2).T


# speedup vs baseline: 1.2620x; 1.0016x over previous
import functools

import jax
import jax.numpy as jnp
from jax import lax
from jax.experimental import pallas as pl
from jax.experimental.pallas import tpu as pltpu
from jax.experimental.pallas import tpu_sc as plsc

NC = 2
NS = 16
NW = NC * NS
LANES = 16


NBUF = 3
UNROLL = 2


def _make_pool(vocab, embed, batch, seq_chunks, chunk):
  rows_per_w = batch // NW
  nreg = embed // LANES
  mesh = plsc.VectorSubcoreMesh(
      core_axis_name="c", subcore_axis_name="s",
      num_cores=NC, num_subcores=NS)

  def body(x_hbm, emb_hbm, out_hbm, idx_v, buf_v, acc_v, *sems):
    wid = lax.axis_index("s") * NC + lax.axis_index("c")
    base = wid * rows_per_w
    pltpu.sync_copy(x_hbm.at[pl.ds(base, rows_per_w)], idx_v)

    def fire(b, p):
      for j in range(seq_chunks):
        pltpu.async_copy(emb_hbm.at[idx_v.at[b, j]], buf_v.at[p, j], sems[p])

    def wait(p):
      for j in range(seq_chunks):
        pltpu.make_async_copy(
            emb_hbm.at[idx_v.at[0, j]], buf_v.at[p, j], sems[p]).wait()

    for p in range(NBUF):
      fire(p, p)

    def accum(p, b):
      def tok(j):
        def f(t, acc):
          for u in range(UNROLL):
            acc = tuple(
                acc[k] + buf_v[p, j, UNROLL * t + u, pl.ds(LANES * k, LANES)]
                for k in range(nreg))
          return acc
        return f
      acc = tuple(jnp.zeros((LANES,), jnp.float32) for _ in range(nreg))
      for j in range(seq_chunks):
        acc = lax.fori_loop(0, chunk // UNROLL, tok(j), acc)
      for k in range(nreg):
        acc_v[b, pl.ds(LANES * k, LANES)] = acc[k]

    main_iters = rows_per_w // NBUF

    def step(i, carry):
      for p in range(NBUF):
        b = NBUF * i + p
        wait(p)
        accum(p, b)

        @pl.when(b + NBUF < rows_per_w)
        def _():
          fire(b + NBUF, p)
      return carry

    lax.fori_loop(0, main_iters, step, 0)
    for b in range(NBUF * main_iters, rows_per_w):
      p = b % NBUF
      wait(p)
      accum(p, b)
    pltpu.sync_copy(acc_v, out_hbm.at[pl.ds(base, rows_per_w)])

  return pl.kernel(
      body,
      out_type=jax.ShapeDtypeStruct((batch, embed), jnp.float32),
      mesh=mesh,
      scratch_types=[
          pltpu.VMEM((rows_per_w, seq_chunks, chunk), jnp.int32),
          pltpu.VMEM((NBUF, seq_chunks, chunk, embed), jnp.float32),
          pltpu.VMEM((rows_per_w, embed), jnp.float32),
      ] + [pltpu.SemaphoreType.DMA] * NBUF,
  )


def _mlp_body(inv_l, p_ref, w1_ref, b1_ref, w2_ref, b2_ref, o_ref):
  pooled = p_ref[:] * inv_l
  h = jnp.maximum(
      jnp.dot(pooled, w1_ref[:], preferred_element_type=jnp.float32)
      + b1_ref[:][None, :], 0.0)
  out_t = lax.dot_general(
      w2_ref[:], h, (((0,), (1,)), ((), ())),
      preferred_element_type=jnp.float32)
  o_ref[:] = out_t + b2_ref[:][:, None]


def kernel(x, embedding, W1, b1, W2, b2):
  batch, seq = x.shape
  vocab, embed = embedding.shape
  hidden = W1.shape[1]
  ncls = W2.shape[1]
  del hidden
  chunk = 100
  seq_chunks = seq // chunk

  xr = x.astype(jnp.int32).reshape(batch, seq_chunks, chunk)
  pool = _make_pool(vocab, embed, batch, seq_chunks, chunk)
  pooled_sum = pool(xr, embedding)

  mlp = pl.pallas_call(
      functools.partial(_mlp_body, 1.0 / seq),
      out_shape=jax.ShapeDtypeStruct((ncls, batch), jnp.float32),
  )
  return mlp(pooled_sum, W1, b1, W2, b2).T


# speedup vs baseline: 1.2631x; 1.0009x over previous
import functools

import jax
import jax.numpy as jnp
from jax import lax
from jax.experimental import pallas as pl
from jax.experimental.pallas import tpu as pltpu
from jax.experimental.pallas import tpu_sc as plsc

NC = 2
NS = 16
NW = NC * NS
LANES = 16


NBUF = 3
UNROLL = 2
ACC_ROWS = 16


def _make_pool(vocab, embed, batch, seq_chunks, chunk):
  rows_per_w = batch // NW
  nreg = embed // LANES
  mesh = plsc.VectorSubcoreMesh(
      core_axis_name="c", subcore_axis_name="s",
      num_cores=NC, num_subcores=NS)

  def body(x_hbm, emb_hbm, out_hbm, idx_v, buf_v, acc_v, *sems):
    wid = lax.axis_index("s") * NC + lax.axis_index("c")
    base = wid * rows_per_w
    pltpu.sync_copy(x_hbm.at[pl.ds(base, rows_per_w)], idx_v)

    def fire(b, p):
      for j in range(seq_chunks):
        pltpu.async_copy(emb_hbm.at[idx_v.at[b, j]], buf_v.at[p, j], sems[p])

    def wait(p):
      for j in range(seq_chunks):
        pltpu.make_async_copy(
            emb_hbm.at[idx_v.at[0, j]], buf_v.at[p, j], sems[p]).wait()

    for p in range(NBUF):
      fire(p, p)

    def accum(p, b):
      def tok(j):
        def f(t, acc):
          for u in range(UNROLL):
            acc = tuple(
                acc[k] + buf_v[p, j, UNROLL * t + u, pl.ds(LANES * k, LANES)]
                for k in range(nreg))
          return acc
        return f
      acc = tuple(jnp.zeros((LANES,), jnp.float32) for _ in range(nreg))
      for j in range(seq_chunks):
        acc = lax.fori_loop(0, chunk // UNROLL, tok(j), acc)
      for k in range(nreg):
        acc_v[b % ACC_ROWS, pl.ds(LANES * k, LANES)] = acc[k]

    main_iters = rows_per_w // NBUF

    def step(i, carry):
      for p in range(NBUF):
        b = NBUF * i + p
        wait(p)
        accum(p, b)

        @pl.when(b + NBUF < rows_per_w)
        def _():
          fire(b + NBUF, p)

        @pl.when((b + 1) % ACC_ROWS == 0)
        def _():
          flush_base = pl.multiple_of(base + b + 1 - ACC_ROWS, ACC_ROWS)
          pltpu.sync_copy(acc_v, out_hbm.at[pl.ds(flush_base, ACC_ROWS)])
      return carry

    lax.fori_loop(0, main_iters, step, 0)
    for b in range(NBUF * main_iters, rows_per_w):
      p = b % NBUF
      wait(p)
      accum(p, b)
    tail_start = (NBUF * main_iters) // ACC_ROWS * ACC_ROWS
    if tail_start < rows_per_w:
      pltpu.sync_copy(acc_v, out_hbm.at[pl.ds(base + tail_start, ACC_ROWS)])

  return pl.kernel(
      body,
      out_type=jax.ShapeDtypeStruct((batch, embed), jnp.float32),
      mesh=mesh,
      scratch_types=[
          pltpu.VMEM((rows_per_w, seq_chunks, chunk), jnp.int32),
          pltpu.VMEM((NBUF, seq_chunks, chunk, embed), jnp.float32),
          pltpu.VMEM((ACC_ROWS, embed), jnp.float32),
      ] + [pltpu.SemaphoreType.DMA] * NBUF,
  )


def _mlp_body(inv_l, p_ref, w1_ref, b1_ref, w2_ref, b2_ref, o_ref):
  pooled = p_ref[:] * inv_l
  h = jnp.maximum(
      jnp.dot(pooled, w1_ref[:], preferred_element_type=jnp.float32)
      + b1_ref[:][None, :], 0.0)
  out_t = lax.dot_general(
      w2_ref[:], h, (((0,), (1,)), ((), ())),
      preferred_element_type=jnp.float32)
  o_ref[:] = out_t + b2_ref[:][:, None]


def kernel(x, embedding, W1, b1, W2, b2):
  batch, seq = x.shape
  vocab, embed = embedding.shape
  hidden = W1.shape[1]
  ncls = W2.shape[1]
  del hidden
  chunk = 100
  seq_chunks = seq // chunk

  xr = x.astype(jnp.int32).reshape(batch, seq_chunks, chunk)
  pool = _make_pool(vocab, embed, batch, seq_chunks, chunk)
  pooled_sum = pool(xr, embedding)

  mlp = pl.pallas_call(
      functools.partial(_mlp_body, 1.0 / seq),
      out_shape=jax.ShapeDtypeStruct((ncls, batch), jnp.float32),
  )
  return mlp(pooled_sum, W1, b1, W2, b2).T
